# baseline (speedup 1.0000x reference)
import jax
import jax.numpy as jnp
from jax import lax
from jax.experimental import pallas as pl
from jax.experimental.pallas import tpu as pltpu
from jax.experimental.pallas import tpu_sc as plsc

NUM_BINS = 381
EMB_DIM = 64
B, L = 16384, 200

_N = B * L
_NC, _NS, _LANES = 2, 16, 16
_NW = _NC * _NS
_PER_W = _N // _NW
_CH = 6400
_NCHUNK = _PER_W // _CH
_NBUF = 3
_BINS_PAD = 384
_LUT_LEN = NUM_BINS * 3
_LUT_PAD = 1152


def _lut_body(t_ref, w1_ref, b1_ref, w2_ref, b2_ref, w3_ref, b3_ref, o_ref):
    t = jnp.concatenate(
        [t_ref[:], jnp.zeros((_BINS_PAD - NUM_BINS, EMB_DIM), jnp.float32)], 0)
    w3 = jnp.concatenate([w3_ref[:], jnp.zeros((32, 125), jnp.float32)], 1)
    b3 = jnp.concatenate([b3_ref[:], jnp.zeros((1, 125), jnp.float32)], 1)
    h = jnp.dot(t, w1_ref[:], preferred_element_type=jnp.float32)
    h = jnp.maximum(h + b1_ref[:], 0.0)
    h = jnp.dot(h, w2_ref[:], preferred_element_type=jnp.float32)
    h = jnp.maximum(h + b2_ref[:], 0.0)
    z = jnp.dot(h, w3, preferred_element_type=jnp.float32) + b3
    o_ref[:] = jax.nn.sigmoid(z)


_lut_call = pl.pallas_call(
    _lut_body,
    out_shape=jax.ShapeDtypeStruct((_BINS_PAD, 128), jnp.float32),
)


def _gather_body(lut_hbm, idx_hbm, amp_hbm, out_hbm, lut_v, *bufs):
    idx_b = bufs[0:_NBUF]
    amp_b = bufs[_NBUF:2 * _NBUF]
    out_b = tuple(bufs[2 * _NBUF + 3 * j:2 * _NBUF + 3 * (j + 1)]
                  for j in range(_NBUF))
    sin = bufs[5 * _NBUF:6 * _NBUF]
    sout = bufs[6 * _NBUF:7 * _NBUF]

    wid = lax.axis_index("s") * _NC + lax.axis_index("c")
    base = wid * _PER_W
    pltpu.sync_copy(lut_hbm, lut_v)

    def start_in(k):
        b = k % _NBUF
        off = base + k * _CH
        h1 = pltpu.async_copy(idx_hbm.at[pl.ds(off, _CH)], idx_b[b], sin[b])
        h2 = pltpu.async_copy(amp_hbm.at[pl.ds(off, _CH)], amp_b[b], sin[b])
        return (h1, h2)

    in_handles = {0: start_in(0), 1: start_in(1)}
    out_handles = {}
    for k in range(_NCHUNK):
        b = k % _NBUF
        if k + 2 < _NCHUNK:
            in_handles[k + 2] = start_in(k + 2)
        for h in in_handles.pop(k):
            h.wait()
        if k >= _NBUF:
            for h in out_handles.pop(k - _NBUF):
                h.wait()

        idx_v, amp_v = idx_b[b], amp_b[b]
        o0, o1, o2 = out_b[b]

        @plsc.parallel_loop(0, _CH // _LANES, unroll=8)
        def vec(i):
            s = i * _LANES
            idx = idx_v[pl.ds(s, _LANES)]
            amp = amp_v[pl.ds(s, _LANES)]
            bright = 0.3 + 0.7 * amp
            g = idx * 3
            o0[pl.ds(s, _LANES)] = plsc.load_gather(lut_v, [g]) * bright
            o1[pl.ds(s, _LANES)] = plsc.load_gather(lut_v, [g + 1]) * bright
            o2[pl.ds(s, _LANES)] = plsc.load_gather(lut_v, [g + 2]) * bright

        off = base + k * _CH
        out_handles[k] = tuple(
            pltpu.async_copy(ov, out_hbm.at[pl.ds(c * _N + off, _CH)], sout[b])
            for c, ov in enumerate((o0, o1, o2)))
    for k in range(_NCHUNK - _NBUF, _NCHUNK):
        for h in out_handles.pop(k):
            h.wait()


_gather_call = pl.kernel(
    _gather_body,
    out_type=jax.ShapeDtypeStruct((3 * _N,), jnp.float32),
    mesh=plsc.VectorSubcoreMesh(core_axis_name="c", subcore_axis_name="s"),
    compiler_params=pltpu.CompilerParams(needs_layout_passes=False),
    scratch_types=(
        [pltpu.VMEM((_LUT_PAD,), jnp.float32)]
        + [pltpu.VMEM((_CH,), jnp.int32) for _ in range(_NBUF)]
        + [pltpu.VMEM((_CH,), jnp.float32) for _ in range(4 * _NBUF)]
        + [pltpu.SemaphoreType.DMA for _ in range(2 * _NBUF)]
    ),
)


def kernel(freq_indices, amplitude, table, W1, b1, W2, b2, W3, b3):
    lut_full = _lut_call(table, W1, b1[None, :], W2, b2[None, :],
                         W3, b3[None, :])
    lut_flat = jnp.pad(lut_full[:NUM_BINS, :3].reshape(-1),
                       (0, _LUT_PAD - _LUT_LEN))
    idx_flat = (freq_indices.T.reshape(L // 8, 8, B // 128, 128)
                .transpose(0, 2, 1, 3).reshape(-1).astype(jnp.int32))
    amp_flat = (amplitude.T.reshape(L // 8, 8, B // 128, 128)
                .transpose(0, 2, 1, 3).reshape(-1))
    out3 = _gather_call(lut_flat, idx_flat, amp_flat)
    return (out3.reshape(3, L // 8, B // 128, 8, 128)
            .transpose(0, 1, 3, 2, 4)
            .reshape(3, L, B)
            .transpose(2, 1, 0))


# speedup vs baseline: 1.0977x; 1.0977x over previous
import jax
import jax.numpy as jnp
from jax import lax
from jax.experimental import pallas as pl
from jax.experimental.pallas import tpu as pltpu
from jax.experimental.pallas import tpu_sc as plsc

NUM_BINS = 381
EMB_DIM = 64
B, L = 16384, 200

_N = B * L
_NC, _NS, _LANES = 2, 16, 16
_NW = _NC * _NS
_PER_W = _N // _NW
_CH = 10240
_NCHUNK = _PER_W // _CH
_NBUF = 2
_BINS_PAD = 384
_LUT_LEN = NUM_BINS * 3
_LUT_PAD = 1152


def _lut_body(tt_ref, w1_ref, b1_ref, w2t_ref, b2_ref, w3t_ref, b3_ref, o_ref):
    h = lax.dot_general(tt_ref[:], w1_ref[:], (((0,), (0,)), ((), ())),
                        preferred_element_type=jnp.float32)
    h = jnp.maximum(h + b1_ref[:], 0.0)
    h = lax.dot_general(h, w2t_ref[:], (((1,), (1,)), ((), ())),
                        preferred_element_type=jnp.float32)
    h = jnp.maximum(h + b2_ref[:], 0.0)
    z = lax.dot_general(h, w3t_ref[:], (((1,), (1,)), ((), ())),
                        preferred_element_type=jnp.float32) + b3_ref[:]
    o_ref[:] = jax.nn.sigmoid(z)


_lut_call = pl.pallas_call(
    _lut_body,
    out_shape=jax.ShapeDtypeStruct((NUM_BINS, 3), jnp.float32),
)


def _gather_body(lut_hbm, idx_hbm, amp_hbm, out_hbm, lut_v, *bufs):
    idx_b = bufs[0:_NBUF]
    amp_b = bufs[_NBUF:2 * _NBUF]
    out_b = tuple(bufs[2 * _NBUF + 3 * j:2 * _NBUF + 3 * (j + 1)]
                  for j in range(_NBUF))
    sin = bufs[5 * _NBUF:6 * _NBUF]
    sout = bufs[6 * _NBUF:7 * _NBUF]

    wid = lax.axis_index("s") * _NC + lax.axis_index("c")
    base = wid * _PER_W
    pltpu.sync_copy(lut_hbm, lut_v)

    def start_in(k):
        b = k % _NBUF
        off = base + k * _CH
        h1 = pltpu.async_copy(idx_hbm.at[pl.ds(off, _CH)], idx_b[b], sin[b])
        h2 = pltpu.async_copy(amp_hbm.at[pl.ds(off, _CH)], amp_b[b], sin[b])
        return (h1, h2)

    in_handles = {0: start_in(0)}
    out_handles = {}
    for k in range(_NCHUNK):
        b = k % _NBUF
        if k + 1 < _NCHUNK:
            in_handles[k + 1] = start_in(k + 1)
        for h in in_handles.pop(k):
            h.wait()
        if k >= _NBUF:
            for h in out_handles.pop(k - _NBUF):
                h.wait()

        idx_v, amp_v = idx_b[b], amp_b[b]
        o0, o1, o2 = out_b[b]

        @plsc.parallel_loop(0, _CH // _LANES, unroll=8)
        def vec(i):
            s = i * _LANES
            idx = idx_v[pl.ds(s, _LANES)]
            amp = amp_v[pl.ds(s, _LANES)]
            bright = 0.3 + 0.7 * amp
            g = idx * 3
            o0[pl.ds(s, _LANES)] = plsc.load_gather(lut_v, [g]) * bright
            o1[pl.ds(s, _LANES)] = plsc.load_gather(lut_v, [g + 1]) * bright
            o2[pl.ds(s, _LANES)] = plsc.load_gather(lut_v, [g + 2]) * bright

        off = base + k * _CH
        out_handles[k] = tuple(
            pltpu.async_copy(ov, out_hbm.at[pl.ds(c * _N + off, _CH)], sout[b])
            for c, ov in enumerate((o0, o1, o2)))
    for k in range(_NCHUNK - _NBUF, _NCHUNK):
        for h in out_handles.pop(k):
            h.wait()


_gather_call = pl.kernel(
    _gather_body,
    out_type=jax.ShapeDtypeStruct((3 * _N,), jnp.float32),
    mesh=plsc.VectorSubcoreMesh(core_axis_name="c", subcore_axis_name="s"),
    compiler_params=pltpu.CompilerParams(needs_layout_passes=False),
    scratch_types=(
        [pltpu.VMEM((_LUT_PAD,), jnp.float32)]
        + [pltpu.VMEM((_CH,), jnp.int32) for _ in range(_NBUF)]
        + [pltpu.VMEM((_CH,), jnp.float32) for _ in range(4 * _NBUF)]
        + [pltpu.SemaphoreType.DMA for _ in range(2 * _NBUF)]
    ),
)


def kernel(freq_indices, amplitude, table, W1, b1, W2, b2, W3, b3):
    lut_full = _lut_call(table.T, W1, b1[None, :], W2.T, b2[None, :],
                         W3.T, b3[None, :])
    lut_flat = jnp.pad(lut_full.reshape(-1), (0, _LUT_PAD - _LUT_LEN))
    idx_flat = (freq_indices.T.reshape(L // 8, 8, B // 128, 128)
                .transpose(0, 2, 1, 3).reshape(-1).astype(jnp.int32))
    amp_flat = (amplitude.T.reshape(L // 8, 8, B // 128, 128)
                .transpose(0, 2, 1, 3).reshape(-1))
    out3 = _gather_call(lut_flat, idx_flat, amp_flat)
    return (out3.reshape(3, L // 8, B // 128, 8, 128)
            .transpose(0, 1, 3, 2, 4)
            .reshape(3, L, B)
            .transpose(2, 1, 0))


# speedup vs baseline: 1.1070x; 1.0084x over previous
import jax
import jax.numpy as jnp
from jax import lax
from jax.experimental import pallas as pl
from jax.experimental.pallas import tpu as pltpu
from jax.experimental.pallas import tpu_sc as plsc

NUM_BINS = 381
EMB_DIM = 64
B, L = 16384, 200

_N = B * L
_NC, _NS, _LANES = 2, 16, 16
_NW = _NC * _NS
_PER_W = _N // _NW
_CH = 10240
_NCHUNK = _PER_W // _CH
_NBUF = 2
_BINS_PAD = 384
_LUT_LEN = NUM_BINS * 3
_LUT_PAD = 1152


def _lut_body(tt_ref, w1_ref, b1_ref, w2t_ref, b2_ref, w3t_ref, b3_ref, o_ref):
    h = lax.dot_general(tt_ref[:], w1_ref[:], (((0,), (0,)), ((), ())),
                        preferred_element_type=jnp.float32)
    h = jnp.maximum(h + b1_ref[:], 0.0)
    h = lax.dot_general(h, w2t_ref[:], (((1,), (1,)), ((), ())),
                        preferred_element_type=jnp.float32)
    h = jnp.maximum(h + b2_ref[:], 0.0)
    z = lax.dot_general(h, w3t_ref[:], (((1,), (1,)), ((), ())),
                        preferred_element_type=jnp.float32) + b3_ref[:]
    o_ref[:] = jax.nn.sigmoid(z)


_lut_call = pl.pallas_call(
    _lut_body,
    out_shape=jax.ShapeDtypeStruct((NUM_BINS, 3), jnp.float32),
)


def _gather_body(lut_hbm, idx_hbm, amp_hbm, out_hbm, lut_v, *bufs):
    idx_b = bufs[0:_NBUF]
    amp_b = bufs[_NBUF:2 * _NBUF]
    out_b = tuple(bufs[2 * _NBUF + 3 * j:2 * _NBUF + 3 * (j + 1)]
                  for j in range(_NBUF))
    sin = bufs[5 * _NBUF:6 * _NBUF]
    sout = bufs[6 * _NBUF:7 * _NBUF]

    wid = lax.axis_index("s") * _NC + lax.axis_index("c")
    base = wid * _PER_W

    def start_in(k):
        b = k % _NBUF
        off = base + k * _CH
        h1 = pltpu.async_copy(idx_hbm.at[pl.ds(off, _CH)], idx_b[b], sin[b])
        h2 = pltpu.async_copy(amp_hbm.at[pl.ds(off, _CH)], amp_b[b], sin[b])
        return (h1, h2)

    in_handles = {0: start_in(0)}
    pltpu.sync_copy(lut_hbm, lut_v)
    out_handles = {}
    for k in range(_NCHUNK):
        b = k % _NBUF
        if k + 1 < _NCHUNK:
            in_handles[k + 1] = start_in(k + 1)
        for h in in_handles.pop(k):
            h.wait()
        if k >= _NBUF:
            for h in out_handles.pop(k - _NBUF):
                h.wait()

        idx_v, amp_v = idx_b[b], amp_b[b]
        o0, o1, o2 = out_b[b]

        @plsc.parallel_loop(0, _CH // _LANES, unroll=10)
        def vec(i):
            s = i * _LANES
            idx = idx_v[pl.ds(s, _LANES)]
            amp = amp_v[pl.ds(s, _LANES)]
            bright = 0.3 + 0.7 * amp
            g = idx * 3
            o0[pl.ds(s, _LANES)] = plsc.load_gather(lut_v, [g]) * bright
            o1[pl.ds(s, _LANES)] = plsc.load_gather(lut_v, [g + 1]) * bright
            o2[pl.ds(s, _LANES)] = plsc.load_gather(lut_v, [g + 2]) * bright

        off = base + k * _CH
        out_handles[k] = tuple(
            pltpu.async_copy(ov, out_hbm.at[pl.ds(c * _N + off, _CH)], sout[b])
            for c, ov in enumerate((o0, o1, o2)))
    for k in range(_NCHUNK - _NBUF, _NCHUNK):
        for h in out_handles.pop(k):
            h.wait()


_gather_call = pl.kernel(
    _gather_body,
    out_type=jax.ShapeDtypeStruct((3 * _N,), jnp.float32),
    mesh=plsc.VectorSubcoreMesh(core_axis_name="c", subcore_axis_name="s"),
    compiler_params=pltpu.CompilerParams(needs_layout_passes=False),
    scratch_types=(
        [pltpu.VMEM((_LUT_PAD,), jnp.float32)]
        + [pltpu.VMEM((_CH,), jnp.int32) for _ in range(_NBUF)]
        + [pltpu.VMEM((_CH,), jnp.float32) for _ in range(4 * _NBUF)]
        + [pltpu.SemaphoreType.DMA for _ in range(2 * _NBUF)]
    ),
)


def kernel(freq_indices, amplitude, table, W1, b1, W2, b2, W3, b3):
    lut_full = _lut_call(table.T, W1, b1[None, :], W2.T, b2[None, :],
                         W3.T, b3[None, :])
    lut_flat = jnp.pad(lut_full.reshape(-1), (0, _LUT_PAD - _LUT_LEN))
    idx_flat = (freq_indices.T.reshape(L // 8, 8, B // 128, 128)
                .transpose(0, 2, 1, 3).reshape(-1).astype(jnp.int32))
    amp_flat = (amplitude.T.reshape(L // 8, 8, B // 128, 128)
                .transpose(0, 2, 1, 3).reshape(-1))
    out3 = _gather_call(lut_flat, idx_flat, amp_flat)
    return (out3.reshape(3, L // 8, B // 128, 8, 128)
            .transpose(0, 1, 3, 2, 4)
            .reshape(3, L, B)
            .transpose(2, 1, 0))
